# baseline (speedup 1.0000x reference)
import functools

import jax
import jax.numpy as jnp
from jax import lax
from jax.experimental import pallas as pl
from jax.experimental.pallas import tpu as pltpu
from jax.experimental.pallas import tpu_sc as plsc

EMB_DIM = 16
NUM_CORES = 2
NUM_SUBCORES = 16
NUM_WORKERS = NUM_CORES * NUM_SUBCORES
CHUNK = 3200


def _sc_gather(x_flat, table):
    total = x_flat.shape[0]
    b_per_w = total // NUM_WORKERS
    nchunk = b_per_w // CHUNK
    mesh = plsc.VectorSubcoreMesh(core_axis_name="c", subcore_axis_name="s")

    @functools.partial(
        pl.kernel,
        mesh=mesh,
        out_type=jax.ShapeDtypeStruct((total, EMB_DIM), jnp.float32),
        scratch_types=[
            pltpu.VMEM((CHUNK,), jnp.int32),
            pltpu.VMEM((CHUNK, EMB_DIM), jnp.float32),
            pltpu.SemaphoreType.DMA,
        ],
        compiler_params=pltpu.CompilerParams(use_tc_tiling_on_sc=False),
    )
    def k(idx_hbm, table_hbm, out_hbm, idx_v, rows_v, sem):
        wid = lax.axis_index("s") * NUM_CORES + lax.axis_index("c")
        base = wid * b_per_w

        def body(i, carry):
            off = base + i * CHUNK
            pltpu.sync_copy(idx_hbm.at[pl.ds(off, CHUNK)], idx_v)
            pltpu.async_copy(table_hbm.at[idx_v], rows_v, sem).wait()
            pltpu.sync_copy(rows_v, out_hbm.at[pl.ds(off, CHUNK)])
            return carry

        lax.fori_loop(0, nchunk, body, 0)

    return k(x_flat, table)


def kernel(x, table):
    x_flat = x.reshape(-1).astype(jnp.int32)
    out = _sc_gather(x_flat, table)
    return out.reshape(x.shape + (EMB_DIM,))


# speedup vs baseline: 1.0084x; 1.0084x over previous
import functools

import jax
import jax.numpy as jnp
from jax import lax
from jax.experimental import pallas as pl
from jax.experimental.pallas import tpu as pltpu
from jax.experimental.pallas import tpu_sc as plsc

EMB_DIM = 16
NUM_CORES = 2
NUM_SUBCORES = 16
NUM_WORKERS = NUM_CORES * NUM_SUBCORES
CHUNK = 2560


def _sc_gather(x3, table):
    nw, nchunk, chunk = x3.shape
    b_per_w = nchunk * chunk
    total = nw * b_per_w
    mesh = plsc.VectorSubcoreMesh(core_axis_name="c", subcore_axis_name="s")

    @functools.partial(
        pl.kernel,
        mesh=mesh,
        out_type=jax.ShapeDtypeStruct((total, EMB_DIM), jnp.float32),
        scratch_types=[
            pltpu.VMEM((nchunk, chunk), jnp.int32),
            pltpu.VMEM((2, chunk, EMB_DIM), jnp.float32),
            pltpu.SemaphoreType.DMA,
            pltpu.SemaphoreType.DMA,
            pltpu.SemaphoreType.DMA,
            pltpu.SemaphoreType.DMA,
        ],
        compiler_params=pltpu.CompilerParams(use_tc_tiling_on_sc=False),
    )
    def k(idx_hbm, table_hbm, out_hbm, idx_all, rows_v, sg0, sg1, sw0, sw1):
        wid = lax.axis_index("s") * NUM_CORES + lax.axis_index("c")
        base = wid * b_per_w
        sg = (sg0, sg1)
        sw = (sw0, sw1)

        pltpu.sync_copy(idx_hbm.at[wid], idx_all)

        g = [None] * nchunk
        w = [None] * nchunk
        g[0] = pltpu.async_copy(table_hbm.at[idx_all.at[0]], rows_v.at[0], sg[0])
        for i in range(nchunk):
            p = i % 2
            if i + 1 < nchunk:
                if i >= 1:
                    w[i - 1].wait()
                g[i + 1] = pltpu.async_copy(
                    table_hbm.at[idx_all.at[i + 1]], rows_v.at[1 - p], sg[1 - p]
                )
            g[i].wait()
            w[i] = pltpu.async_copy(
                rows_v.at[p], out_hbm.at[pl.ds(base + i * chunk, chunk)], sw[p]
            )
        if nchunk >= 2:
            w[nchunk - 2].wait()
        w[nchunk - 1].wait()

    return k(x3, table)


def kernel(x, table):
    total = x.shape[0] * x.shape[1]
    b_per_w = total // NUM_WORKERS
    nchunk = b_per_w // CHUNK
    x3 = x.reshape(NUM_WORKERS, nchunk, CHUNK).astype(jnp.int32)
    out = _sc_gather(x3, table)
    return out.reshape(x.shape + (EMB_DIM,))


# speedup vs baseline: 1.0407x; 1.0320x over previous
import functools

import jax
import jax.numpy as jnp
from jax import lax
from jax.experimental import pallas as pl
from jax.experimental.pallas import tpu as pltpu
from jax.experimental.pallas import tpu_sc as plsc

EMB_DIM = 16
NUM_CORES = 2
NUM_SUBCORES = 16
NUM_WORKERS = NUM_CORES * NUM_SUBCORES
CH = 128
GROUPS = CH // 16


def _sc_gather(x2, table2):
    nw, b_per_w = x2.shape
    sup_rows = table2.shape[0]
    nch = b_per_w // CH
    total = nw * b_per_w
    mesh = plsc.VectorSubcoreMesh(core_axis_name="c", subcore_axis_name="s")

    @functools.partial(
        pl.kernel,
        mesh=mesh,
        out_type=jax.ShapeDtypeStruct((nw, b_per_w * EMB_DIM), jnp.float32),
        scratch_types=[
            pltpu.VMEM((b_per_w,), jnp.int32),
            pltpu.VMEM((CH, 128), jnp.float32),
            pltpu.VMEM((CH, 128), jnp.float32),
            pltpu.VMEM((CH * EMB_DIM,), jnp.float32),
            pltpu.VMEM((CH * EMB_DIM,), jnp.float32),
            pltpu.VMEM((CH,), jnp.int32),
            pltpu.VMEM((CH,), jnp.int32),
            pltpu.VMEM((CH,), jnp.int32),
            pltpu.VMEM((CH,), jnp.int32),
            pltpu.SemaphoreType.DMA,
            pltpu.SemaphoreType.DMA,
            pltpu.SemaphoreType.DMA,
            pltpu.SemaphoreType.DMA,
        ],
        compiler_params=pltpu.CompilerParams(use_tc_tiling_on_sc=False, needs_layout_passes=False),
    )
    def k(idx_hbm, table_hbm, out_hbm, idx_v, gat0, gat1, cmp0, cmp1,
          sup0, sup1, off0, off1, sg0, sg1, sw0, sw1):
        wid = lax.axis_index("s") * NUM_CORES + lax.axis_index("c")
        gat = (gat0, gat1)
        cmp = (cmp0, cmp1)
        sup = (sup0, sup1)
        off = (off0, off1)
        sg = (sg0, sg1)
        sw = (sw0, sw1)
        iota = lax.iota(jnp.int32, 16)

        pltpu.sync_copy(idx_hbm.at[wid], idx_v)

        def prep(c, p):
            for g in range(GROUPS):
                fi = c * CH + g * 16
                idx16 = idx_v[pl.ds(fi, 16)]
                sup[p][pl.ds(g * 16, 16)] = lax.shift_right_logical(idx16, 3)
                off[p][pl.ds(g * 16, 16)] = lax.shift_left(
                    lax.bitwise_and(idx16, 7), 4)

        def extract(p):
            for g in range(GROUPS):
                rows16 = g * 16 + iota
                rbase = lax.shift_left(rows16, 4)
                off16 = off[p][pl.ds(g * 16, 16)]
                for col in range(16):
                    vals = plsc.load_gather(gat[p], [rows16, off16 + col])
                    plsc.store_scatter(cmp[p], [rbase + col], vals)

        def start_gather(c, p):
            return pltpu.async_copy(table_hbm.at[sup[p]], gat[p], sg[p])

        def wait_gather(p):
            pltpu.make_async_copy(table_hbm.at[sup[p]], gat[p], sg[p]).wait()

        def start_wb(c, p):
            return pltpu.async_copy(
                cmp[p], out_hbm.at[wid, pl.ds(c * (CH * EMB_DIM), CH * EMB_DIM)],
                sw[p])

        def wait_wb(p):
            pltpu.make_async_copy(
                cmp[p], out_hbm.at[wid, pl.ds(0, CH * EMB_DIM)], sw[p]).wait()

        def body(i, carry):
            for b in range(2):
                c = 2 * i + b

                @pl.when(i >= 1)
                def _():
                    wait_gather(b)

                @pl.when(i >= 2)
                def _():
                    wait_wb(b)

                @pl.when(i >= 1)
                def _():
                    extract(b)
                    start_wb(c - 2, b)

                prep(c, b)
                start_gather(c, b)
            return carry

        lax.fori_loop(0, nch // 2, body, 0)

        for b in range(2):
            c = nch - 2 + b
            wait_gather(b)
            wait_wb(b)
            extract(b)
            start_wb(c, b)
        for b in range(2):
            wait_wb(b)

    return k(x2, table2)


def kernel(x, table):
    total = x.shape[0] * x.shape[1]
    b_per_w = total // NUM_WORKERS
    x2 = x.reshape(NUM_WORKERS, b_per_w).astype(jnp.int32)
    table2 = table.reshape(table.shape[0] // 8, 128)
    out = _sc_gather(x2, table2)
    return out.reshape(x.shape + (EMB_DIM,))
